# baseline (speedup 1.0000x reference)
import functools

import jax
import jax.numpy as jnp
from jax import lax
from jax.experimental import pallas as pl
from jax.experimental.pallas import tpu as pltpu
from jax.experimental.pallas import tpu_sc as plsc

DIM = 64
_IDXW = 128
_CH = 512
_KS = _CH // _IDXW


def _sc_gather(table, idx2d):
    n_rows_total = idx2d.shape[0]
    B = n_rows_total * _IDXW
    info = plsc.get_sparse_core_info()
    NC, NS = info.num_cores, info.num_subcores
    NW = NC * NS
    b_per_w = B // NW
    n_ch = b_per_w // _CH
    rows_per_ch = _KS
    mesh = plsc.VectorSubcoreMesh(core_axis_name="c", subcore_axis_name="s")

    @functools.partial(
        pl.kernel,
        out_type=jax.ShapeDtypeStruct((B, DIM), jnp.float32),
        mesh=mesh,
        compiler_params=pltpu.CompilerParams(use_tc_tiling_on_sc=False),
        scratch_types=[
            pltpu.VMEM((_KS, _IDXW), jnp.int32),
            pltpu.VMEM((_CH, DIM), jnp.float32),
            pltpu.SemaphoreType.DMA,
        ],
    )
    def gather_k(table_hbm, idx_hbm, out_hbm, idx_v, rows_v, gsem):
        wid = lax.axis_index("s") * NC + lax.axis_index("c")
        row_base = wid * (b_per_w // _IDXW)
        out_base = wid * b_per_w

        def body(g, _):
            pltpu.sync_copy(
                idx_hbm.at[pl.ds(row_base + g * rows_per_ch, rows_per_ch)],
                idx_v,
            )
            for j in range(_KS):
                pltpu.async_copy(
                    table_hbm.at[idx_v.at[j]],
                    rows_v.at[pl.ds(j * _IDXW, _IDXW)],
                    gsem,
                )
            for j in range(_KS):
                pltpu.make_async_copy(
                    table_hbm.at[idx_v.at[j]],
                    rows_v.at[pl.ds(j * _IDXW, _IDXW)],
                    gsem,
                ).wait()
            pltpu.sync_copy(
                rows_v,
                out_hbm.at[pl.ds(out_base + g * _CH, _CH)],
            )
            return _

        lax.fori_loop(0, n_ch, body, None)

    return gather_k(table, idx2d)


def _tc_project(x, sigma, Vt, rv):
    B = x.shape[0]
    BT = 8192
    grid = B // BT

    def proj_k(x_ref, s_ref, vt_ref, rv_ref, o_ref):
        y = jnp.dot(x_ref[...] * s_ref[...], vt_ref[...],
                    preferred_element_type=jnp.float32)
        for i in range(rv.shape[0]):
            v = rv_ref[i:i + 1, :]
            vn = v / (jnp.sqrt(jnp.sum(v * v)) + 1e-12)
            y = y + jnp.sum(y * vn, axis=1, keepdims=True) * v
        o_ref[...] = y

    return pl.pallas_call(
        proj_k,
        grid=(grid,),
        in_specs=[
            pl.BlockSpec((BT, DIM), lambda i: (i, 0)),
            pl.BlockSpec((1, DIM), lambda i: (0, 0)),
            pl.BlockSpec((DIM, DIM), lambda i: (0, 0)),
            pl.BlockSpec((rv.shape[0], DIM), lambda i: (0, 0)),
        ],
        out_specs=pl.BlockSpec((BT, DIM), lambda i: (i, 0)),
        out_shape=jax.ShapeDtypeStruct((B, DIM), jnp.float32),
    )(x, sigma.reshape(1, DIM), Vt, rv)


def kernel(indices, U, sigma, Vt, right_vecs):
    Bo, L = indices.shape
    B = Bo * L
    idx2d = indices.reshape(B // _IDXW, _IDXW).astype(jnp.int32)
    gathered = _sc_gather(U, idx2d)
    out = _tc_project(gathered, sigma, Vt, right_vecs)
    return out.reshape(Bo, L, DIM)


# speedup vs baseline: 1.0271x; 1.0271x over previous
import functools

import jax
import jax.numpy as jnp
from jax import lax
from jax.experimental import pallas as pl
from jax.experimental.pallas import tpu as pltpu
from jax.experimental.pallas import tpu_sc as plsc

DIM = 64
_IDXW = 128
_CH = 512
_KS = _CH // _IDXW


_RPC = 8


def _sc_gather(table, indices):
    NR, L = indices.shape
    B = NR * L
    _p1 = min(128, -(-(L // 2) // 8) * 8)
    parts = ((0, _p1), (_p1, L - _p1))
    info = plsc.get_sparse_core_info()
    NC, NS = info.num_cores, info.num_subcores
    NW = NC * NS
    r_per_w = NR // NW
    n_ch = r_per_w // _RPC
    mesh = plsc.VectorSubcoreMesh(core_axis_name="c", subcore_axis_name="s")

    @functools.partial(
        pl.kernel,
        out_type=jax.ShapeDtypeStruct((B, DIM), jnp.float32),
        mesh=mesh,
        compiler_params=pltpu.CompilerParams(use_tc_tiling_on_sc=False),
        scratch_types=[
            pltpu.VMEM((_RPC, L), jnp.int32),
            pltpu.VMEM((_RPC * L, DIM), jnp.float32),
            pltpu.SemaphoreType.DMA,
        ],
    )
    def gather_k(table_hbm, idx_hbm, out_hbm, idx_v, rows_v, gsem):
        wid = lax.axis_index("s") * NC + lax.axis_index("c")
        row_base = wid * r_per_w

        def body(g, _):
            r0 = row_base + g * _RPC
            pltpu.sync_copy(idx_hbm.at[pl.ds(r0, _RPC)], idx_v)
            for r in range(_RPC):
                for off, sz in parts:
                    pltpu.async_copy(
                        table_hbm.at[idx_v.at[r, pl.ds(off, sz)]],
                        rows_v.at[pl.ds(r * L + off, sz)],
                        gsem,
                    )
            for r in range(_RPC):
                for off, sz in parts:
                    pltpu.make_async_copy(
                        table_hbm.at[idx_v.at[r, pl.ds(off, sz)]],
                        rows_v.at[pl.ds(r * L + off, sz)],
                        gsem,
                    ).wait()
            pltpu.sync_copy(rows_v, out_hbm.at[pl.ds(r0 * L, _RPC * L)])
            return _

        lax.fori_loop(0, n_ch, body, None)

    return gather_k(table, indices)


def _tc_project(x, sigma, Vt, rv):
    B = x.shape[0]
    BT = 8192
    grid = B // BT

    def proj_k(x_ref, s_ref, vt_ref, rv_ref, o_ref):
        y = jnp.dot(x_ref[...] * s_ref[...], vt_ref[...],
                    preferred_element_type=jnp.float32)
        for i in range(rv.shape[0]):
            v = rv_ref[i:i + 1, :]
            vn = v / (jnp.sqrt(jnp.sum(v * v)) + 1e-12)
            y = y + jnp.sum(y * vn, axis=1, keepdims=True) * v
        o_ref[...] = y

    return pl.pallas_call(
        proj_k,
        grid=(grid,),
        in_specs=[
            pl.BlockSpec((BT, DIM), lambda i: (i, 0)),
            pl.BlockSpec((1, DIM), lambda i: (0, 0)),
            pl.BlockSpec((DIM, DIM), lambda i: (0, 0)),
            pl.BlockSpec((rv.shape[0], DIM), lambda i: (0, 0)),
        ],
        out_specs=pl.BlockSpec((BT, DIM), lambda i: (i, 0)),
        out_shape=jax.ShapeDtypeStruct((B, DIM), jnp.float32),
    )(x, sigma.reshape(1, DIM), Vt, rv)


def kernel(indices, U, sigma, Vt, right_vecs):
    Bo, L = indices.shape
    gathered = _sc_gather(U, indices.astype(jnp.int32))
    out = _tc_project(gathered, sigma, Vt, right_vecs)
    return out.reshape(Bo, L, DIM)
